# baseline (speedup 1.0000x reference)
import functools

import jax
import jax.numpy as jnp
from jax import lax
from jax.experimental import pallas as pl
from jax.experimental.pallas import tpu as pltpu
from jax.experimental.pallas import tpu_sc as plsc

N_FACTORS = 32
BATCH = 16384
NC, NS, L = 2, 16, 16
NW = NC * NS
BPW = BATCH // NW
CHUNK = 128
NCHUNK = BPW // CHUNK
NGROUP = BPW // L


def _body(data_hbm, uf_hbm, if_hbm, out_hbm,
          data_v, uidx, iidx, urows, irows, out_v, sem):
  wid = lax.axis_index("s") * NC + lax.axis_index("c")
  base = wid * BPW

  pltpu.sync_copy(data_hbm.at[pl.ds(2 * base, 2 * BPW)], data_v)

  iot = lax.iota(jnp.int32, L)

  for g in range(NGROUP):
    rows2 = (iot + iot) + (2 * g * L)
    u_i = plsc.load_gather(data_v, [rows2])
    v_i = plsc.load_gather(data_v, [rows2 + 1])
    j, off = g // (CHUNK // L), (g % (CHUNK // L)) * L
    uidx[j, pl.ds(off, L)] = u_i
    iidx[j, pl.ds(off, L)] = v_i

  copies = []
  for j in range(NCHUNK):
    copies.append(pltpu.async_copy(
        uf_hbm.at[uidx.at[j]], urows.at[pl.ds(j * CHUNK, CHUNK), :], sem))
    copies.append(pltpu.async_copy(
        if_hbm.at[iidx.at[j]], irows.at[pl.ds(j * CHUNK, CHUNK), :], sem))
  for c in copies:
    c.wait()

  def group_body(g, carry):
    rows = iot + g * L
    acc = jnp.zeros((L,), jnp.float32)
    for d in range(N_FACTORS):
      dcol = jnp.full((L,), d, jnp.int32)
      uu = plsc.load_gather(urows, [rows, dcol])
      vv = plsc.load_gather(irows, [rows, dcol])
      acc = acc + uu * vv
    out_v[pl.ds(g * L, L)] = acc
    return carry

  lax.fori_loop(0, NGROUP, group_body, 0)

  pltpu.sync_copy(out_v, out_hbm.at[pl.ds(base, BPW)])


_mesh = plsc.VectorSubcoreMesh(core_axis_name="c", subcore_axis_name="s")

_sc_call = functools.partial(
    pl.kernel,
    out_type=jax.ShapeDtypeStruct((BATCH,), jnp.float32),
    mesh=_mesh,
    compiler_params=pltpu.CompilerParams(
        needs_layout_passes=False, use_tc_tiling_on_sc=False),
    scratch_types=[
        pltpu.VMEM((2 * BPW,), jnp.int32),
        pltpu.VMEM((NCHUNK, CHUNK), jnp.int32),
        pltpu.VMEM((NCHUNK, CHUNK), jnp.int32),
        pltpu.VMEM((BPW, N_FACTORS), jnp.float32),
        pltpu.VMEM((BPW, N_FACTORS), jnp.float32),
        pltpu.VMEM((BPW,), jnp.float32),
        pltpu.SemaphoreType.DMA,
    ],
)(_body)


@jax.jit
def kernel(data, user_factors, item_factors):
  flat = data.astype(jnp.int32).reshape(-1)
  return _sc_call(flat, user_factors, item_factors)


# speedup vs baseline: 3.3758x; 3.3758x over previous
import functools

import jax
import jax.numpy as jnp
from jax import lax
from jax.experimental import pallas as pl
from jax.experimental.pallas import tpu as pltpu
from jax.experimental.pallas import tpu_sc as plsc

N_FACTORS = 32
BATCH = 16384
NC, NS, L = 2, 16, 16
NW = NC * NS
BPW = BATCH // NW
NGROUP = BPW // L
TCOL = 128
SB = 4
NSB = L // SB


def _body(data_hbm, uf_hbm, if_hbm, out_hbm,
          data_v, ustart, istart, ulane, ilane,
          ubuf, vbuf, pbuf, out_v, sem_u0, sem_u1, sem_v0, sem_v1):
  wid = lax.axis_index("s") * NC + lax.axis_index("c")
  base = wid * BPW

  pltpu.sync_copy(data_hbm.at[pl.ds(2 * base, 2 * BPW)], data_v)

  iot = lax.iota(jnp.int32, L)
  zeros = jnp.zeros((L,), jnp.int32)

  for g in range(NGROUP):
    rows2 = (iot + iot) + (2 * g * L)
    u_i = plsc.load_gather(data_v, [rows2])
    v_i = plsc.load_gather(data_v, [rows2 + 1])
    s = pl.ds(g * L, L)
    ustart[s] = u_i - (u_i & (TCOL - 1))
    ulane[s] = u_i & (TCOL - 1)
    istart[s] = v_i - (v_i & (TCOL - 1))
    ilane[s] = v_i & (TCOL - 1)

  u_sems = (sem_u0, sem_u1)
  v_sems = (sem_v0, sem_v1)

  def fire(suv, siv, sb):
    p = sb % 2
    for k in range(SB):
      su = pl.multiple_of(suv[SB * sb + k], TCOL)
      si = pl.multiple_of(siv[SB * sb + k], TCOL)
      pltpu.async_copy(uf_hbm.at[:, pl.ds(su, TCOL)], ubuf.at[p, k],
                       u_sems[p])
      pltpu.async_copy(if_hbm.at[:, pl.ds(si, TCOL)], vbuf.at[p, k],
                       v_sems[p])

  def drain(sb):
    p = sb % 2
    for k in range(SB):
      pltpu.make_async_copy(uf_hbm.at[:, pl.ds(0, TCOL)], ubuf.at[p, k],
                            u_sems[p]).wait()
      pltpu.make_async_copy(if_hbm.at[:, pl.ds(0, TCOL)], vbuf.at[p, k],
                            v_sems[p]).wait()

  def compute(quv, qvv, sb):
    p = sb % 2
    for k in range(SB):
      qu = quv[SB * sb + k] + zeros
      qv = qvv[SB * sb + k] + zeros
      ul = plsc.load_gather(ubuf.at[p, k], [iot, qu])
      uh = plsc.load_gather(ubuf.at[p, k], [iot + L, qu])
      vl = plsc.load_gather(vbuf.at[p, k], [iot, qv])
      vh = plsc.load_gather(vbuf.at[p, k], [iot + L, qv])
      pbuf[SB * sb + k, :] = ul * vl + uh * vh

  def group_body(g, carry):
    s = pl.ds(g * L, L)
    suv = ustart[s]
    siv = istart[s]
    quv = ulane[s]
    qvv = ilane[s]
    fire(suv, siv, 0)
    fire(suv, siv, 1)
    for sb in range(NSB):
      drain(sb)
      if sb + 2 < NSB:
        fire(suv, siv, sb + 2)
      compute(quv, qvv, sb)
    accv = jnp.zeros((L,), jnp.float32)
    for d in range(L):
      dcol = jnp.full((L,), d, jnp.int32)
      accv = accv + plsc.load_gather(pbuf, [iot, dcol])
    out_v[s] = accv
    return carry

  lax.fori_loop(0, NGROUP, group_body, 0)

  pltpu.sync_copy(out_v, out_hbm.at[pl.ds(base, BPW)])


_mesh = plsc.VectorSubcoreMesh(core_axis_name="c", subcore_axis_name="s")

_sc_call = functools.partial(
    pl.kernel,
    out_type=jax.ShapeDtypeStruct((BATCH,), jnp.float32),
    mesh=_mesh,
    compiler_params=pltpu.CompilerParams(needs_layout_passes=False),
    scratch_types=[
        pltpu.VMEM((2 * BPW,), jnp.int32),
        pltpu.VMEM((BPW,), jnp.int32),
        pltpu.VMEM((BPW,), jnp.int32),
        pltpu.VMEM((BPW,), jnp.int32),
        pltpu.VMEM((BPW,), jnp.int32),
        pltpu.VMEM((2, SB, N_FACTORS, TCOL), jnp.float32),
        pltpu.VMEM((2, SB, N_FACTORS, TCOL), jnp.float32),
        pltpu.VMEM((L, L), jnp.float32),
        pltpu.VMEM((BPW,), jnp.float32),
        pltpu.SemaphoreType.DMA,
        pltpu.SemaphoreType.DMA,
        pltpu.SemaphoreType.DMA,
        pltpu.SemaphoreType.DMA,
    ],
)(_body)


@jax.jit
def kernel(data, user_factors, item_factors):
  flat = data.astype(jnp.int32).reshape(-1)
  return _sc_call(flat, user_factors.T, item_factors.T)
